# baseline (speedup 1.0000x reference)
import functools

import jax
import jax.numpy as jnp
from jax import lax
from jax.experimental import pallas as pl
from jax.experimental.pallas import tpu as pltpu
from jax.experimental.pallas import tpu_sc as plsc

D_MODEL = 64
SCALE = 0.125

NC = 2
NS = 16
NW = NC * NS
LANES = 16
CHUNK = 512


def _make_kernel(B):
    b_per_w = B // NW
    n_chunks = b_per_w // CHUNK
    assert B % NW == 0 and b_per_w % CHUNK == 0 and n_chunks % 2 == 0

    mesh = plsc.VectorSubcoreMesh(core_axis_name="c", subcore_axis_name="s")

    @functools.partial(
        pl.kernel,
        out_type=jax.ShapeDtypeStruct((B, D_MODEL), jnp.float32),
        mesh=mesh,
        compiler_params=pltpu.CompilerParams(use_tc_tiling_on_sc=False),
        scratch_types=[
            pltpu.VMEM((b_per_w,), jnp.int32),
            pltpu.VMEM((2, CHUNK, D_MODEL), jnp.float32),
            pltpu.SemaphoreType.DMA,
            pltpu.SemaphoreType.DMA,
            pltpu.SemaphoreType.DMA,
            pltpu.SemaphoreType.DMA,
        ],
    )
    def emb_kernel(x_hbm, table_hbm, out_hbm, idx_v, rows_v, g0, g1, o0, o1):
        gsems = (g0, g1)
        osems = (o0, o1)
        wid = lax.axis_index("s") * NC + lax.axis_index("c")
        base = wid * b_per_w

        pltpu.sync_copy(x_hbm.at[pl.ds(base, b_per_w)], idx_v)

        def start_gather(j, s):
            pltpu.async_copy(
                table_hbm.at[idx_v.at[pl.ds(j * CHUNK, CHUNK)]],
                rows_v.at[s], gsems[s])

        def wait_gather(s):
            pltpu.make_async_copy(
                table_hbm.at[idx_v.at[pl.ds(0, CHUNK)]],
                rows_v.at[s], gsems[s]).wait()

        def start_store(j, s):
            pltpu.async_copy(
                rows_v.at[s],
                out_hbm.at[pl.ds(base + j * CHUNK, CHUNK)], osems[s])

        def wait_store(s):
            pltpu.make_async_copy(
                rows_v.at[s],
                out_hbm.at[pl.ds(base, CHUNK)], osems[s]).wait()

        def scale(s):
            @pl.loop(0, CHUNK)
            def _(r):
                for c in range(D_MODEL // LANES):
                    col = pl.ds(c * LANES, LANES)
                    rows_v[s, r, col] = rows_v[s, r, col] * SCALE

        start_gather(0, 0)

        @pl.loop(0, n_chunks, step=2)
        def _(i):
            @pl.when(i > 0)
            def _():
                wait_store(1)
            start_gather(i + 1, 1)
            wait_gather(0)
            scale(0)
            start_store(i, 0)

            @pl.when(i + 2 < n_chunks)
            def _():
                wait_store(0)
                start_gather(i + 2, 0)
            wait_gather(1)
            scale(1)
            start_store(i + 1, 1)

        wait_store(0)
        wait_store(1)

    return emb_kernel


def kernel(x, table):
    orig_shape = x.shape
    xf = x.reshape(-1).astype(jnp.int32)
    out = _make_kernel(xf.shape[0])(xf, table)
    return out.reshape(*orig_shape, D_MODEL)


# speedup vs baseline: 1.0260x; 1.0260x over previous
import functools

import jax
import jax.numpy as jnp
from jax import lax
from jax.experimental import pallas as pl
from jax.experimental.pallas import tpu as pltpu
from jax.experimental.pallas import tpu_sc as plsc

D_MODEL = 64
SCALE = 0.125

NC = 2
NS = 16
NW = NC * NS
LANES = 16
CHUNK = 512


def _make_kernel(B):
    b_per_w = B // NW
    n_chunks = b_per_w // CHUNK
    assert B % NW == 0 and b_per_w % CHUNK == 0 and n_chunks % 2 == 0

    mesh = plsc.VectorSubcoreMesh(core_axis_name="c", subcore_axis_name="s")

    @functools.partial(
        pl.kernel,
        out_type=jax.ShapeDtypeStruct((B, D_MODEL), jnp.float32),
        mesh=mesh,
        compiler_params=pltpu.CompilerParams(use_tc_tiling_on_sc=False),
        scratch_types=[
            pltpu.VMEM((b_per_w,), jnp.int32),
            pltpu.VMEM((2, CHUNK, D_MODEL), jnp.float32),
            pltpu.SemaphoreType.DMA,
            pltpu.SemaphoreType.DMA,
            pltpu.SemaphoreType.DMA,
            pltpu.SemaphoreType.DMA,
        ],
    )
    def emb_kernel(x_hbm, table_hbm, out_hbm, idx_v, rows_v, g0, g1, o0, o1):
        gsems = (g0, g1)
        osems = (o0, o1)
        wid = lax.axis_index("s") * NC + lax.axis_index("c")
        base = wid * b_per_w

        pltpu.sync_copy(x_hbm.at[pl.ds(base, b_per_w)], idx_v)

        def start_gather(j, s):
            pltpu.async_copy(
                table_hbm.at[idx_v.at[pl.ds(j * CHUNK, CHUNK)]],
                rows_v.at[s], gsems[s])

        def wait_gather(s):
            pltpu.make_async_copy(
                table_hbm.at[idx_v.at[pl.ds(0, CHUNK)]],
                rows_v.at[s], gsems[s]).wait()

        def start_store(j, s):
            pltpu.async_copy(
                rows_v.at[s],
                out_hbm.at[pl.ds(base + j * CHUNK, CHUNK)], osems[s])

        def wait_store(s):
            pltpu.make_async_copy(
                rows_v.at[s],
                out_hbm.at[pl.ds(base, CHUNK)], osems[s]).wait()

        def scale(s):
            @pl.loop(0, CHUNK, unroll=8)
            def _(r):
                for c in range(D_MODEL // LANES):
                    col = pl.ds(c * LANES, LANES)
                    rows_v[s, r, col] = rows_v[s, r, col] * SCALE

        start_gather(0, 0)

        @pl.loop(0, n_chunks, step=2)
        def _(i):
            @pl.when(i > 0)
            def _():
                wait_store(1)
            start_gather(i + 1, 1)
            wait_gather(0)
            scale(0)
            start_store(i, 0)

            @pl.when(i + 2 < n_chunks)
            def _():
                wait_store(0)
                start_gather(i + 2, 0)
            wait_gather(1)
            scale(1)
            start_store(i + 1, 1)

        wait_store(0)
        wait_store(1)

    return emb_kernel


def kernel(x, table):
    orig_shape = x.shape
    xf = x.reshape(-1).astype(jnp.int32)
    out = _make_kernel(xf.shape[0])(xf, table)
    return out.reshape(*orig_shape, D_MODEL)
